# Initial kernel scaffold; baseline (speedup 1.0000x reference)
import jax
import jax.numpy as jnp
from jax.experimental import pallas as pl


def kernel(init_x, x, x_bond, edge_index_intra, edge_index_inter, pos, W_coord_cov, b_coord_cov, W_coord_ncov, b_coord_ncov, W_bond, b_bond, W_node_cov, b_node_cov, W_node_ncov, b_node_ncov):
    raise NotImplementedError("write your pallas kernel here")


# speedup vs baseline: 1.6089x; 1.6089x over previous
import functools

import numpy as np
import jax
import jax.numpy as jnp
from jax import lax
from jax.experimental import pallas as pl
from jax.experimental.pallas import tpu as pltpu
from jax.experimental.pallas import tpu_sc as plsc

NC = 2
NS = 16
NW = NC * NS
BC = 64


def _silu(v):
    return v / (1.0 + jnp.exp(-v))


def _lrelu(v):
    return jnp.where(v >= 0, v, 0.1 * v)


def _chunk(n, cap=128):
    for cand in range(min(n, cap), 0, -1):
        if n % cand == 0 and cand % 8 == 0:
            return cand
    return 8


def _build_dist_kernel(n_nodes, epc, epn):
    eac, ean = epc // NW, epn // NW
    ebuf = max(eac, ean)
    mesh = plsc.VectorSubcoreMesh(core_axis_name="c", subcore_axis_name="s")

    @functools.partial(
        pl.kernel,
        out_type=[jax.ShapeDtypeStruct((epc,), jnp.float32),
                  jax.ShapeDtypeStruct((epn,), jnp.float32)],
        mesh=mesh,
        scratch_types=[
            pltpu.VMEM((n_nodes,), jnp.float32),
            pltpu.VMEM((n_nodes,), jnp.float32),
            pltpu.VMEM((n_nodes,), jnp.float32),
            pltpu.VMEM((ebuf,), jnp.int32),
            pltpu.VMEM((ebuf,), jnp.int32),
            pltpu.VMEM((ebuf,), jnp.float32),
        ],
        compiler_params=pltpu.CompilerParams(needs_layout_passes=False),
    )
    def dist_k(px_h, py_h, pz_h, sc_h, dc_h, sn_h, dn_h, d2c_h, d2n_h,
               px_v, py_v, pz_v, s_v, t_v, d2_v):
        c = lax.axis_index("c")
        s = lax.axis_index("s")
        wid = s * NC + c
        pltpu.sync_copy(px_h, px_v)
        pltpu.sync_copy(py_h, py_v)
        pltpu.sync_copy(pz_h, pz_v)

        def one(src_h, dst_h, out_h, ea):
            base = wid * ea
            pltpu.sync_copy(src_h.at[pl.ds(base, ea)], s_v.at[pl.ds(0, ea)])
            pltpu.sync_copy(dst_h.at[pl.ds(base, ea)], t_v.at[pl.ds(0, ea)])

            def body(j, carry):
                sl = pl.ds(j * 16, 16)
                si = s_v[sl]
                di = t_v[sl]
                dx = plsc.load_gather(px_v, [di]) - plsc.load_gather(px_v, [si])
                dy = plsc.load_gather(py_v, [di]) - plsc.load_gather(py_v, [si])
                dz = plsc.load_gather(pz_v, [di]) - plsc.load_gather(pz_v, [si])
                d2_v[sl] = dx * dx + dy * dy + dz * dz
                return carry

            lax.fori_loop(0, ea // 16, body, 0)
            pltpu.sync_copy(d2_v.at[pl.ds(0, ea)], out_h.at[pl.ds(base, ea)])

        one(sc_h, dc_h, d2c_h, eac)
        one(sn_h, dn_h, d2n_h, ean)

    return dist_k


def _build_edge_kernel(li, n_pad, h_dim, epc, epn, n_layers):
    eac, ean = epc // NW, epn // NW
    npt = n_pad // NS
    nd = _chunk(npt)
    mesh = plsc.VectorSubcoreMesh(core_axis_name="c", subcore_axis_name="s")

    @functools.partial(
        pl.kernel,
        out_type=[jax.ShapeDtypeStruct((NC, n_pad, h_dim), jnp.float32),
                  jax.ShapeDtypeStruct((NC, n_pad, h_dim), jnp.float32)],
        mesh=mesh,
        scratch_types=[
            pltpu.VMEM_SHARED((n_pad, h_dim), jnp.float32),
            pltpu.VMEM((BC,), jnp.int32),
            pltpu.VMEM((BC,), jnp.int32),
            pltpu.VMEM((BC, h_dim), jnp.float32),
            pltpu.VMEM((BC, h_dim), jnp.float32),
            pltpu.VMEM((BC, h_dim), jnp.float32),
            pltpu.VMEM((nd, h_dim), jnp.float32),
            pltpu.SemaphoreType.DMA,
        ],
    )
    def edge_k(h_hbm, radc_h, radn_h, bond_h, sc_h, dc_h, sn_h, dn_h,
               aggc_h, aggn_h,
               agg_sh, si_v, di_v, rows_v, rad_v, bond_v, tmp_v, sem):
        c = lax.axis_index("c")
        s = lax.axis_index("s")
        wid = s * NC + c
        row0 = s * npt

        def zero_tmp():
            def zb(r, carry):
                for g in range(h_dim // 16):
                    tmp_v[r, pl.ds(g * 16, 16)] = jnp.zeros((16,), jnp.float32)
                return carry
            lax.fori_loop(0, nd, zb, 0)

        def zero_agg():
            for k in range(npt // nd):
                pltpu.sync_copy(tmp_v, agg_sh.at[pl.ds(row0 + k * nd, nd), :])

        def edge_pass(src_h, dst_h, rad_h, with_bond, ea):
            base = wid * ea

            def blk(b, carry):
                off = base + b * BC
                pltpu.sync_copy(src_h.at[pl.ds(off, BC)], si_v)
                pltpu.sync_copy(dst_h.at[pl.ds(off, BC)], di_v)
                pltpu.async_copy(h_hbm.at[si_v], rows_v, sem).wait()
                pltpu.sync_copy(rad_h.at[li, pl.ds(off, BC), :], rad_v)
                if with_bond:
                    pltpu.sync_copy(bond_h.at[li, pl.ds(off, BC), :], bond_v)

                def rb(r, cc):
                    for g in range(h_dim // 16):
                        sl = pl.ds(g * 16, 16)
                        v = rows_v[r, sl]
                        if with_bond:
                            v = v + bond_v[r, sl]
                        rows_v[r, sl] = v * rad_v[r, sl]
                    return cc

                lax.fori_loop(0, BC, rb, 0)
                pltpu.sync_copy(rows_v, agg_sh.at[di_v], add=True)
                return carry

            lax.fori_loop(0, ea // BC, blk, 0)

        def dump(out_h):
            for k in range(npt // nd):
                pltpu.sync_copy(agg_sh.at[pl.ds(row0 + k * nd, nd), :], tmp_v)
                pltpu.sync_copy(tmp_v, out_h.at[c, pl.ds(row0 + k * nd, nd), :])

        zero_tmp()
        zero_agg()
        plsc.subcore_barrier()
        edge_pass(sc_h, dc_h, radc_h, True, eac)
        plsc.subcore_barrier()
        dump(aggc_h)
        zero_tmp()
        zero_agg()
        plsc.subcore_barrier()
        edge_pass(sn_h, dn_h, radn_h, False, ean)
        plsc.subcore_barrier()
        dump(aggn_h)

    return edge_k


def _radial_bond_tc(d2c, d2n, xb, wcc, bcc, wcn, bcn, wb, bb,
                    ec, en, n_layers, dc, h_dim, be=2048):
    ep = d2c.shape[0]
    grid = (ep // be, n_layers)
    step_c = np.float32(6.0 / (dc - 1))
    step_n = np.float32(10.0 / (dc - 1))
    inv_sig_c = np.float32(dc / 6.0)
    inv_sig_n = np.float32(dc / 10.0)

    def body(d2c_ref, d2n_ref, xb_ref, wcc_ref, bcc_ref, wcn_ref, bcn_ref,
             wb_ref, bb_ref, rc_ref, rn_ref, bd_ref):
        e = pl.program_id(0)
        rows = e * be + lax.broadcasted_iota(jnp.int32, (be, 1), 0)

        def radial(d2_ref, w_ref, b_ref, mu_step, inv_sig, limit):
            mu = (lax.broadcasted_iota(jnp.int32, (1, dc), 1)
                  .astype(jnp.float32) * mu_step)
            d = jnp.sqrt(d2_ref[...] + 1e-12)
            t = (d - mu) * inv_sig
            rbf = jnp.exp(-(t * t))
            r = jnp.dot(rbf, w_ref[0], preferred_element_type=jnp.float32)
            r = _silu(r + b_ref[0])
            return jnp.where(rows < limit, r, 0.0)

        rc_ref[0] = radial(d2c_ref, wcc_ref, bcc_ref, step_c, inv_sig_c, ec)
        rn_ref[0] = radial(d2n_ref, wcn_ref, bcn_ref, step_n, inv_sig_n, en)
        bd_ref[0] = (jnp.dot(xb_ref[...], wb_ref[0],
                             preferred_element_type=jnp.float32) + bb_ref[0])

    return pl.pallas_call(
        body,
        grid=grid,
        in_specs=[
            pl.BlockSpec((be, 1), lambda e, i: (e, 0)),
            pl.BlockSpec((be, 1), lambda e, i: (e, 0)),
            pl.BlockSpec((be, h_dim), lambda e, i: (e, 0)),
            pl.BlockSpec((1, dc, h_dim), lambda e, i: (i, 0, 0)),
            pl.BlockSpec((1, 1, h_dim), lambda e, i: (i, 0, 0)),
            pl.BlockSpec((1, dc, h_dim), lambda e, i: (i, 0, 0)),
            pl.BlockSpec((1, 1, h_dim), lambda e, i: (i, 0, 0)),
            pl.BlockSpec((1, h_dim, h_dim), lambda e, i: (i, 0, 0)),
            pl.BlockSpec((1, 1, h_dim), lambda e, i: (i, 0, 0)),
        ],
        out_specs=[
            pl.BlockSpec((1, be, h_dim), lambda e, i: (i, e, 0)),
            pl.BlockSpec((1, be, h_dim), lambda e, i: (i, e, 0)),
            pl.BlockSpec((1, be, h_dim), lambda e, i: (i, e, 0)),
        ],
        out_shape=[
            jax.ShapeDtypeStruct((n_layers, ep, h_dim), jnp.float32),
            jax.ShapeDtypeStruct((n_layers, ep, h_dim), jnp.float32),
            jax.ShapeDtypeStruct((n_layers, ep, h_dim), jnp.float32),
        ],
    )(d2c, d2n, xb, wcc, bcc[:, None], wcn, bcn[:, None], wb, bb[:, None])


def _node_tc(h, aggc, aggn, wc, bc, wn, bn, x0, bn_rows=2000):
    n_nodes, h_dim = h.shape
    grid = (n_nodes // bn_rows,)

    def body(h_ref, ac_ref, an_ref, wc_ref, bc_ref, wn_ref, bn_ref, x0_ref,
             o_ref):
        hh = h_ref[...]
        zc = jnp.dot(hh + ac_ref[0] + ac_ref[1], wc_ref[...],
                     preferred_element_type=jnp.float32) + bc_ref[...]
        zn = jnp.dot(hh + an_ref[0] + an_ref[1], wn_ref[...],
                     preferred_element_type=jnp.float32) + bn_ref[...]
        o_ref[...] = _lrelu(zc) + _lrelu(zn) + x0_ref[...]

    return pl.pallas_call(
        body,
        grid=grid,
        in_specs=[
            pl.BlockSpec((bn_rows, h_dim), lambda e: (e, 0)),
            pl.BlockSpec((2, bn_rows, h_dim), lambda e: (0, e, 0)),
            pl.BlockSpec((2, bn_rows, h_dim), lambda e: (0, e, 0)),
            pl.BlockSpec((h_dim, h_dim), lambda e: (0, 0)),
            pl.BlockSpec((1, h_dim), lambda e: (0, 0)),
            pl.BlockSpec((h_dim, h_dim), lambda e: (0, 0)),
            pl.BlockSpec((1, h_dim), lambda e: (0, 0)),
            pl.BlockSpec((bn_rows, h_dim), lambda e: (e, 0)),
        ],
        out_specs=pl.BlockSpec((bn_rows, h_dim), lambda e: (e, 0)),
        out_shape=jax.ShapeDtypeStruct((n_nodes, h_dim), jnp.float32),
    )(h, aggc, aggn, wc, bc[None], wn, bn[None], x0)


def kernel(init_x, x, x_bond, edge_index_intra, edge_index_inter, pos,
           W_coord_cov, b_coord_cov, W_coord_ncov, b_coord_ncov,
           W_bond, b_bond, W_node_cov, b_node_cov, W_node_ncov, b_node_ncov):
    n_nodes, h_dim = x.shape
    ec = edge_index_intra.shape[1]
    en = edge_index_inter.shape[1]
    n_layers = W_bond.shape[0]
    dc = W_coord_cov.shape[1]

    quant = NW * BC
    ep = ((max(ec, en) + quant - 1) // quant) * quant
    nq = NS * 8
    n_pad = ((n_nodes + nq - 1) // nq) * nq

    src_c = jnp.pad(edge_index_intra[0].astype(jnp.int32), (0, ep - ec))
    dst_c = jnp.pad(edge_index_intra[1].astype(jnp.int32), (0, ep - ec))
    src_n = jnp.pad(edge_index_inter[0].astype(jnp.int32), (0, ep - en))
    dst_n = jnp.pad(edge_index_inter[1].astype(jnp.int32), (0, ep - en))
    xb = jnp.pad(x_bond, ((0, ep - ec), (0, 0)))
    px = pos[:, 0] + 0.0
    py = pos[:, 1] + 0.0
    pz = pos[:, 2] + 0.0

    d2c, d2n = _build_dist_kernel(n_nodes, ep, ep)(
        px, py, pz, src_c, dst_c, src_n, dst_n)

    rc, rn, bd = _radial_bond_tc(
        d2c.reshape(ep, 1), d2n.reshape(ep, 1), xb,
        W_coord_cov, b_coord_cov, W_coord_ncov, b_coord_ncov,
        W_bond, b_bond, ec, en, n_layers, dc, h_dim)

    h = x
    for i in range(n_layers):
        aggc, aggn = _build_edge_kernel(i, n_pad, h_dim, ep, ep, n_layers)(
            h, rc, rn, bd, src_c, dst_c, src_n, dst_n)
        h = _node_tc(h, aggc, aggn, W_node_cov[i], b_node_cov[i],
                     W_node_ncov[i], b_node_ncov[i], init_x)
    return h


# speedup vs baseline: 2.6685x; 1.6587x over previous
import functools

import numpy as np
import jax
import jax.numpy as jnp
from jax import lax
from jax.experimental import pallas as pl
from jax.experimental.pallas import tpu as pltpu
from jax.experimental.pallas import tpu_sc as plsc

NC = 2
NS = 16
NW = NC * NS
BC = 64


def _silu(v):
    return v / (1.0 + jnp.exp(-v))


def _lrelu(v):
    return jnp.where(v >= 0, v, 0.1 * v)


def _chunk(n, cap=128):
    for cand in range(min(n, cap), 0, -1):
        if n % cand == 0 and cand % 8 == 0:
            return cand
    return 8


def _build_dist_kernel(n_nodes, epc, epn):
    eac, ean = epc // NW, epn // NW
    ebuf = max(eac, ean)
    mesh = plsc.VectorSubcoreMesh(core_axis_name="c", subcore_axis_name="s")

    @functools.partial(
        pl.kernel,
        out_type=[jax.ShapeDtypeStruct((epc,), jnp.float32),
                  jax.ShapeDtypeStruct((epn,), jnp.float32)],
        mesh=mesh,
        scratch_types=[
            pltpu.VMEM((n_nodes,), jnp.float32),
            pltpu.VMEM((n_nodes,), jnp.float32),
            pltpu.VMEM((n_nodes,), jnp.float32),
            pltpu.VMEM((ebuf,), jnp.int32),
            pltpu.VMEM((ebuf,), jnp.int32),
            pltpu.VMEM((ebuf,), jnp.float32),
        ],
        compiler_params=pltpu.CompilerParams(needs_layout_passes=False),
    )
    def dist_k(px_h, py_h, pz_h, sc_h, dc_h, sn_h, dn_h, d2c_h, d2n_h,
               px_v, py_v, pz_v, s_v, t_v, d2_v):
        c = lax.axis_index("c")
        s = lax.axis_index("s")
        wid = s * NC + c
        pltpu.sync_copy(px_h, px_v)
        pltpu.sync_copy(py_h, py_v)
        pltpu.sync_copy(pz_h, pz_v)

        def one(src_h, dst_h, out_h, ea):
            base = wid * ea
            pltpu.sync_copy(src_h.at[pl.ds(base, ea)], s_v.at[pl.ds(0, ea)])
            pltpu.sync_copy(dst_h.at[pl.ds(base, ea)], t_v.at[pl.ds(0, ea)])

            def body(j, carry):
                sl = pl.ds(j * 16, 16)
                si = s_v[sl]
                di = t_v[sl]
                dx = plsc.load_gather(px_v, [di]) - plsc.load_gather(px_v, [si])
                dy = plsc.load_gather(py_v, [di]) - plsc.load_gather(py_v, [si])
                dz = plsc.load_gather(pz_v, [di]) - plsc.load_gather(pz_v, [si])
                d2_v[sl] = dx * dx + dy * dy + dz * dz
                return carry

            lax.fori_loop(0, ea // 16, body, 0)
            pltpu.sync_copy(d2_v.at[pl.ds(0, ea)], out_h.at[pl.ds(base, ea)])

        one(sc_h, dc_h, d2c_h, eac)
        one(sn_h, dn_h, d2n_h, ean)

    return dist_k


def _build_edge_kernel(li, n_pad, h_dim, epc, epn, n_layers):
    epw = epc // NS
    nb = epw // BC
    assert epn == epc and nb % 2 == 0
    npt = n_pad // NS
    ndz = 40
    chunks = []
    o = 0
    while o < npt:
        chunks.append((o, min(ndz, npt - o)))
        o += ndz
    mesh = plsc.VectorSubcoreMesh(core_axis_name="c", subcore_axis_name="s")

    @functools.partial(
        pl.kernel,
        out_type=[jax.ShapeDtypeStruct((n_pad, h_dim), jnp.float32),
                  jax.ShapeDtypeStruct((n_pad, h_dim), jnp.float32)],
        mesh=mesh,
        scratch_types=[
            pltpu.VMEM_SHARED((n_pad, h_dim), jnp.float32),
            pltpu.VMEM((BC,), jnp.int32),
            pltpu.VMEM((BC,), jnp.int32),
            pltpu.VMEM((BC,), jnp.int32),
            pltpu.VMEM((BC,), jnp.int32),
            pltpu.VMEM((BC, h_dim), jnp.float32),
            pltpu.VMEM((BC, h_dim), jnp.float32),
            pltpu.VMEM((BC, h_dim), jnp.float32),
            pltpu.VMEM((BC, h_dim), jnp.float32),
            pltpu.VMEM((ndz, h_dim), jnp.float32),
            pltpu.SemaphoreType.DMA,
            pltpu.SemaphoreType.DMA,
            pltpu.SemaphoreType.DMA,
            pltpu.SemaphoreType.DMA,
            pltpu.SemaphoreType.DMA,
            pltpu.SemaphoreType.DMA,
            pltpu.SemaphoreType.DMA,
            pltpu.SemaphoreType.DMA,
            pltpu.SemaphoreType.DMA,
            pltpu.SemaphoreType.DMA,
        ],
    )
    def edge_k(h_hbm, radc_h, radn_h, bond_h, sc_h, dc_h, sn_h, dn_h,
               aggc_h, aggn_h,
               agg_sh, si0, si1, di0, di1, rows0, rows1, rad0, rad1, bnc,
               smi0, smi1, smb0, smb1, smr0, smr1, smg0, smg1, sms0, sms1):
        c = lax.axis_index("c")
        s = lax.axis_index("s")
        base = s * epw
        row0 = s * npt
        si = (si0, si1)
        di = (di0, di1)
        rows = (rows0, rows1)
        rad = (rad0, rad1)
        smi = (smi0, smi1)
        smb = (smb0, smb1)
        smr = (smr0, smr1)
        smg = (smg0, smg1)
        sms = (sms0, sms1)

        def zero_bounce():
            def zb(r, carry):
                for g in range(h_dim // 16):
                    bnc[r, pl.ds(g * 16, 16)] = jnp.zeros((16,), jnp.float32)
                return carry
            lax.fori_loop(0, ndz, zb, 0)

        def zero_agg():
            for (ro, rn) in chunks:
                pltpu.sync_copy(bnc.at[pl.ds(0, rn), :],
                                agg_sh.at[pl.ds(row0 + ro, rn), :])

        def dump(out_h):
            for (ro, rn) in chunks:
                pltpu.sync_copy(agg_sh.at[pl.ds(row0 + ro, rn), :],
                                bnc.at[pl.ds(0, rn), :])
                pltpu.sync_copy(bnc.at[pl.ds(0, rn), :],
                                out_h.at[pl.ds(row0 + ro, rn), :])

        def one_pass(src_h, dst_h, rad_h, use_bond, out_h):
            def issue_loads(p, b):
                off = base + b * BC
                pltpu.async_copy(src_h.at[pl.ds(off, BC)], si[p], smi[p])
                pltpu.async_copy(dst_h.at[pl.ds(off, BC)], di[p], smi[p])
                if use_bond:
                    pltpu.async_copy(bond_h.at[li, pl.ds(off, BC), :],
                                     rows[p], smb[p])
                pltpu.async_copy(rad_h.at[li, pl.ds(off, BC), :],
                                 rad[p], smr[p])

            def wait_idx(p):
                pltpu.make_async_copy(src_h.at[pl.ds(0, BC)], si[p],
                                      smi[p]).wait()
                pltpu.make_async_copy(dst_h.at[pl.ds(0, BC)], di[p],
                                      smi[p]).wait()

            def start_gather(p):
                wait_idx(p)
                if use_bond:
                    pltpu.make_async_copy(bond_h.at[li, pl.ds(0, BC), :],
                                          rows[p], smb[p]).wait()
                    pltpu.async_copy(h_hbm.at[si[p]], rows[p], smg[p],
                                     add=True)
                else:
                    pltpu.async_copy(h_hbm.at[si[p]], rows[p], smg[p])

            def compute_scatter(p):
                pltpu.make_async_copy(h_hbm.at[si[p]], rows[p],
                                      smg[p]).wait()
                pltpu.make_async_copy(rad_h.at[li, pl.ds(0, BC), :], rad[p],
                                      smr[p]).wait()

                def rb(r, cc):
                    for g in range(h_dim // 16):
                        sl = pl.ds(g * 16, 16)
                        rows[p][r, sl] = rows[p][r, sl] * rad[p][r, sl]
                    return cc

                lax.fori_loop(0, BC, rb, 0)
                pltpu.async_copy(rows[p], agg_sh.at[di[p]], sms[p], add=True)

            def wait_scatter(p):
                pltpu.make_async_copy(rows[p], agg_sh.at[di[p]],
                                      sms[p]).wait()

            zero_bounce()
            zero_agg()
            issue_loads(0, 0)
            issue_loads(1, 1)
            plsc.subcore_barrier()

            def body(k, carry):
                a = 2 * k
                start_gather(0)
                start_gather(1)
                compute_scatter(0)
                compute_scatter(1)
                wait_scatter(0)
                issue_loads(0, a + 2)
                wait_scatter(1)
                issue_loads(1, a + 3)
                return carry

            lax.fori_loop(0, nb // 2 - 1, body, 0)
            start_gather(0)
            start_gather(1)
            compute_scatter(0)
            compute_scatter(1)
            wait_scatter(0)
            wait_scatter(1)
            plsc.subcore_barrier()
            dump(out_h)

        @pl.when(c == 0)
        def _():
            one_pass(sc_h, dc_h, radc_h, True, aggc_h)

        @pl.when(c == 1)
        def _():
            one_pass(sn_h, dn_h, radn_h, False, aggn_h)

    return edge_k


def _radial_bond_tc(d2c, d2n, xb, wcc, bcc, wcn, bcn, wb, bb,
                    ec, en, n_layers, dc, h_dim, be=2048):
    ep = d2c.shape[0]
    grid = (ep // be, n_layers)
    step_c = np.float32(6.0 / (dc - 1))
    step_n = np.float32(10.0 / (dc - 1))
    inv_sig_c = np.float32(dc / 6.0)
    inv_sig_n = np.float32(dc / 10.0)

    def body(d2c_ref, d2n_ref, xb_ref, wcc_ref, bcc_ref, wcn_ref, bcn_ref,
             wb_ref, bb_ref, rc_ref, rn_ref, bd_ref):
        e = pl.program_id(0)
        rows = e * be + lax.broadcasted_iota(jnp.int32, (be, 1), 0)

        def radial(d2_ref, w_ref, b_ref, mu_step, inv_sig, limit):
            mu = (lax.broadcasted_iota(jnp.int32, (1, dc), 1)
                  .astype(jnp.float32) * mu_step)
            d = jnp.sqrt(d2_ref[...] + 1e-12)
            t = (d - mu) * inv_sig
            rbf = jnp.exp(-(t * t))
            r = jnp.dot(rbf, w_ref[0], preferred_element_type=jnp.float32)
            r = _silu(r + b_ref[0])
            return jnp.where(rows < limit, r, 0.0)

        rc_ref[0] = radial(d2c_ref, wcc_ref, bcc_ref, step_c, inv_sig_c, ec)
        rn_ref[0] = radial(d2n_ref, wcn_ref, bcn_ref, step_n, inv_sig_n, en)
        bd_ref[0] = (jnp.dot(xb_ref[...], wb_ref[0],
                             preferred_element_type=jnp.float32) + bb_ref[0])

    return pl.pallas_call(
        body,
        grid=grid,
        in_specs=[
            pl.BlockSpec((be, 1), lambda e, i: (e, 0)),
            pl.BlockSpec((be, 1), lambda e, i: (e, 0)),
            pl.BlockSpec((be, h_dim), lambda e, i: (e, 0)),
            pl.BlockSpec((1, dc, h_dim), lambda e, i: (i, 0, 0)),
            pl.BlockSpec((1, 1, h_dim), lambda e, i: (i, 0, 0)),
            pl.BlockSpec((1, dc, h_dim), lambda e, i: (i, 0, 0)),
            pl.BlockSpec((1, 1, h_dim), lambda e, i: (i, 0, 0)),
            pl.BlockSpec((1, h_dim, h_dim), lambda e, i: (i, 0, 0)),
            pl.BlockSpec((1, 1, h_dim), lambda e, i: (i, 0, 0)),
        ],
        out_specs=[
            pl.BlockSpec((1, be, h_dim), lambda e, i: (i, e, 0)),
            pl.BlockSpec((1, be, h_dim), lambda e, i: (i, e, 0)),
            pl.BlockSpec((1, be, h_dim), lambda e, i: (i, e, 0)),
        ],
        out_shape=[
            jax.ShapeDtypeStruct((n_layers, ep, h_dim), jnp.float32),
            jax.ShapeDtypeStruct((n_layers, ep, h_dim), jnp.float32),
            jax.ShapeDtypeStruct((n_layers, ep, h_dim), jnp.float32),
        ],
    )(d2c, d2n, xb, wcc, bcc[:, None], wcn, bcn[:, None], wb, bb[:, None])


def _node_tc(h, aggc, aggn, wc, bc, wn, bn, x0, bn_rows=2000):
    n_nodes, h_dim = h.shape
    grid = (n_nodes // bn_rows,)

    def body(h_ref, ac_ref, an_ref, wc_ref, bc_ref, wn_ref, bn_ref, x0_ref,
             o_ref):
        hh = h_ref[...]
        zc = jnp.dot(hh + ac_ref[...], wc_ref[...],
                     preferred_element_type=jnp.float32) + bc_ref[...]
        zn = jnp.dot(hh + an_ref[...], wn_ref[...],
                     preferred_element_type=jnp.float32) + bn_ref[...]
        o_ref[...] = _lrelu(zc) + _lrelu(zn) + x0_ref[...]

    return pl.pallas_call(
        body,
        grid=grid,
        in_specs=[
            pl.BlockSpec((bn_rows, h_dim), lambda e: (e, 0)),
            pl.BlockSpec((bn_rows, h_dim), lambda e: (e, 0)),
            pl.BlockSpec((bn_rows, h_dim), lambda e: (e, 0)),
            pl.BlockSpec((h_dim, h_dim), lambda e: (0, 0)),
            pl.BlockSpec((1, h_dim), lambda e: (0, 0)),
            pl.BlockSpec((h_dim, h_dim), lambda e: (0, 0)),
            pl.BlockSpec((1, h_dim), lambda e: (0, 0)),
            pl.BlockSpec((bn_rows, h_dim), lambda e: (e, 0)),
        ],
        out_specs=pl.BlockSpec((bn_rows, h_dim), lambda e: (e, 0)),
        out_shape=jax.ShapeDtypeStruct((n_nodes, h_dim), jnp.float32),
    )(h, aggc, aggn, wc, bc[None], wn, bn[None], x0)


def kernel(init_x, x, x_bond, edge_index_intra, edge_index_inter, pos,
           W_coord_cov, b_coord_cov, W_coord_ncov, b_coord_ncov,
           W_bond, b_bond, W_node_cov, b_node_cov, W_node_ncov, b_node_ncov):
    n_nodes, h_dim = x.shape
    ec = edge_index_intra.shape[1]
    en = edge_index_inter.shape[1]
    n_layers = W_bond.shape[0]
    dc = W_coord_cov.shape[1]

    quant = NW * BC
    ep = ((max(ec, en) + quant - 1) // quant) * quant
    nq = NS * 8
    n_pad = ((n_nodes + nq - 1) // nq) * nq

    src_c = jnp.pad(edge_index_intra[0].astype(jnp.int32), (0, ep - ec))
    dst_c = jnp.pad(edge_index_intra[1].astype(jnp.int32), (0, ep - ec))
    src_n = jnp.pad(edge_index_inter[0].astype(jnp.int32), (0, ep - en))
    dst_n = jnp.pad(edge_index_inter[1].astype(jnp.int32), (0, ep - en))
    xb = jnp.pad(x_bond, ((0, ep - ec), (0, 0)))
    px = pos[:, 0] + 0.0
    py = pos[:, 1] + 0.0
    pz = pos[:, 2] + 0.0

    d2c, d2n = _build_dist_kernel(n_nodes, ep, ep)(
        px, py, pz, src_c, dst_c, src_n, dst_n)

    rc, rn, bd = _radial_bond_tc(
        d2c.reshape(ep, 1), d2n.reshape(ep, 1), xb,
        W_coord_cov, b_coord_cov, W_coord_ncov, b_coord_ncov,
        W_bond, b_bond, ec, en, n_layers, dc, h_dim)

    h = x
    for i in range(n_layers):
        aggc, aggn = _build_edge_kernel(i, n_pad, h_dim, ep, ep, n_layers)(
            h, rc, rn, bd, src_c, dst_c, src_n, dst_n)
        h = _node_tc(h, aggc, aggn, W_node_cov[i], b_node_cov[i],
                     W_node_ncov[i], b_node_ncov[i], init_x)
    return h


# speedup vs baseline: 2.9717x; 1.1136x over previous
import functools

import numpy as np
import jax
import jax.numpy as jnp
from jax import lax
from jax.experimental import pallas as pl
from jax.experimental.pallas import tpu as pltpu
from jax.experimental.pallas import tpu_sc as plsc

NC = 2
NS = 16
NW = NC * NS
BC = 64


def _silu(v):
    return v / (1.0 + jnp.exp(-v))


def _lrelu(v):
    return jnp.where(v >= 0, v, 0.1 * v)


def _chunk(n, cap=128):
    for cand in range(min(n, cap), 0, -1):
        if n % cand == 0 and cand % 8 == 0:
            return cand
    return 8


def _build_dist_kernel(n_nodes, epc, epn):
    eac, ean = epc // NW, epn // NW
    ebuf = max(eac, ean)
    mesh = plsc.VectorSubcoreMesh(core_axis_name="c", subcore_axis_name="s")

    @functools.partial(
        pl.kernel,
        out_type=[jax.ShapeDtypeStruct((epc,), jnp.float32),
                  jax.ShapeDtypeStruct((epn,), jnp.float32)],
        mesh=mesh,
        scratch_types=[
            pltpu.VMEM((n_nodes,), jnp.float32),
            pltpu.VMEM((n_nodes,), jnp.float32),
            pltpu.VMEM((n_nodes,), jnp.float32),
            pltpu.VMEM((ebuf,), jnp.int32),
            pltpu.VMEM((ebuf,), jnp.int32),
            pltpu.VMEM((ebuf,), jnp.float32),
        ],
        compiler_params=pltpu.CompilerParams(needs_layout_passes=False),
    )
    def dist_k(px_h, py_h, pz_h, sc_h, dc_h, sn_h, dn_h, d2c_h, d2n_h,
               px_v, py_v, pz_v, s_v, t_v, d2_v):
        c = lax.axis_index("c")
        s = lax.axis_index("s")
        wid = s * NC + c
        pltpu.sync_copy(px_h, px_v)
        pltpu.sync_copy(py_h, py_v)
        pltpu.sync_copy(pz_h, pz_v)

        def one(src_h, dst_h, out_h, ea):
            base = wid * ea
            pltpu.sync_copy(src_h.at[pl.ds(base, ea)], s_v.at[pl.ds(0, ea)])
            pltpu.sync_copy(dst_h.at[pl.ds(base, ea)], t_v.at[pl.ds(0, ea)])

            def body(j, carry):
                sl = pl.ds(j * 16, 16)
                si = s_v[sl]
                di = t_v[sl]
                dx = plsc.load_gather(px_v, [di]) - plsc.load_gather(px_v, [si])
                dy = plsc.load_gather(py_v, [di]) - plsc.load_gather(py_v, [si])
                dz = plsc.load_gather(pz_v, [di]) - plsc.load_gather(pz_v, [si])
                d2_v[sl] = dx * dx + dy * dy + dz * dz
                return carry

            lax.fori_loop(0, ea // 16, body, 0)
            pltpu.sync_copy(d2_v.at[pl.ds(0, ea)], out_h.at[pl.ds(base, ea)])

        one(sc_h, dc_h, d2c_h, eac)
        one(sn_h, dn_h, d2n_h, ean)

    return dist_k


def _build_edge_kernel(n_pad, h_dim, epc, epn):
    epw = epc // NS
    nb = epw // BC
    assert epn == epc and nb % 2 == 0
    npt = n_pad // NS
    ndz = 40
    chunks = []
    o = 0
    while o < npt:
        chunks.append((o, min(ndz, npt - o)))
        o += ndz
    mesh = plsc.VectorSubcoreMesh(core_axis_name="c", subcore_axis_name="s")

    @functools.partial(
        pl.kernel,
        out_type=[jax.ShapeDtypeStruct((n_pad, h_dim), jnp.float32),
                  jax.ShapeDtypeStruct((n_pad, h_dim), jnp.float32)],
        mesh=mesh,
        scratch_types=[
            pltpu.VMEM_SHARED((n_pad, h_dim), jnp.float32),
            pltpu.VMEM((BC,), jnp.int32),
            pltpu.VMEM((BC,), jnp.int32),
            pltpu.VMEM((BC,), jnp.int32),
            pltpu.VMEM((BC,), jnp.int32),
            pltpu.VMEM((BC, h_dim), jnp.float32),
            pltpu.VMEM((BC, h_dim), jnp.float32),
            pltpu.VMEM((BC, h_dim), jnp.float32),
            pltpu.VMEM((BC, h_dim), jnp.float32),
            pltpu.VMEM((ndz, h_dim), jnp.float32),
            pltpu.SemaphoreType.DMA,
            pltpu.SemaphoreType.DMA,
            pltpu.SemaphoreType.DMA,
            pltpu.SemaphoreType.DMA,
            pltpu.SemaphoreType.DMA,
            pltpu.SemaphoreType.DMA,
            pltpu.SemaphoreType.DMA,
            pltpu.SemaphoreType.DMA,
            pltpu.SemaphoreType.DMA,
            pltpu.SemaphoreType.DMA,
        ],
    )
    def edge_k(h_hbm, radc_h, radn_h, bond_h, sc_h, dc_h, sn_h, dn_h,
               aggc_h, aggn_h,
               agg_sh, si0, si1, di0, di1, rows0, rows1, rad0, rad1, bnc,
               smi0, smi1, smb0, smb1, smr0, smr1, smg0, smg1, sms0, sms1):
        c = lax.axis_index("c")
        s = lax.axis_index("s")
        base = s * epw
        row0 = s * npt
        si = (si0, si1)
        di = (di0, di1)
        rows = (rows0, rows1)
        rad = (rad0, rad1)
        smi = (smi0, smi1)
        smb = (smb0, smb1)
        smr = (smr0, smr1)
        smg = (smg0, smg1)
        sms = (sms0, sms1)

        def zero_bounce():
            def zb(r, carry):
                for g in range(h_dim // 16):
                    bnc[r, pl.ds(g * 16, 16)] = jnp.zeros((16,), jnp.float32)
                return carry
            lax.fori_loop(0, ndz, zb, 0)

        def zero_agg():
            for (ro, rn) in chunks:
                pltpu.sync_copy(bnc.at[pl.ds(0, rn), :],
                                agg_sh.at[pl.ds(row0 + ro, rn), :])

        def dump(out_h):
            for (ro, rn) in chunks:
                pltpu.sync_copy(agg_sh.at[pl.ds(row0 + ro, rn), :],
                                bnc.at[pl.ds(0, rn), :])
                pltpu.sync_copy(bnc.at[pl.ds(0, rn), :],
                                out_h.at[pl.ds(row0 + ro, rn), :])

        def one_pass(src_h, dst_h, rad_h, use_bond, out_h):
            def issue_loads(p, b):
                off = base + b * BC
                pltpu.async_copy(src_h.at[pl.ds(off, BC)], si[p], smi[p])
                pltpu.async_copy(dst_h.at[pl.ds(off, BC)], di[p], smi[p])
                if use_bond:
                    pltpu.async_copy(bond_h.at[pl.ds(off, BC), :],
                                     rows[p], smb[p])
                pltpu.async_copy(rad_h.at[pl.ds(off, BC), :],
                                 rad[p], smr[p])

            def wait_idx(p):
                pltpu.make_async_copy(src_h.at[pl.ds(0, BC)], si[p],
                                      smi[p]).wait()
                pltpu.make_async_copy(dst_h.at[pl.ds(0, BC)], di[p],
                                      smi[p]).wait()

            def start_gather(p):
                wait_idx(p)
                if use_bond:
                    pltpu.make_async_copy(bond_h.at[pl.ds(0, BC), :],
                                          rows[p], smb[p]).wait()
                    pltpu.async_copy(h_hbm.at[si[p]], rows[p], smg[p],
                                     add=True)
                else:
                    pltpu.async_copy(h_hbm.at[si[p]], rows[p], smg[p])

            def compute_scatter(p):
                pltpu.make_async_copy(h_hbm.at[si[p]], rows[p],
                                      smg[p]).wait()
                pltpu.make_async_copy(rad_h.at[pl.ds(0, BC), :], rad[p],
                                      smr[p]).wait()

                def rb(r, cc):
                    for g in range(h_dim // 16):
                        sl = pl.ds(g * 16, 16)
                        rows[p][r, sl] = rows[p][r, sl] * rad[p][r, sl]
                    return cc

                lax.fori_loop(0, BC, rb, 0)
                pltpu.async_copy(rows[p], agg_sh.at[di[p]], sms[p], add=True)

            def wait_scatter(p):
                pltpu.make_async_copy(rows[p], agg_sh.at[di[p]],
                                      sms[p]).wait()

            zero_bounce()
            zero_agg()
            issue_loads(0, 0)
            issue_loads(1, 1)
            plsc.subcore_barrier()

            def body(k, carry):
                a = 2 * k
                start_gather(0)
                start_gather(1)
                compute_scatter(0)
                compute_scatter(1)
                wait_scatter(0)
                issue_loads(0, a + 2)
                wait_scatter(1)
                issue_loads(1, a + 3)
                return carry

            lax.fori_loop(0, nb // 2 - 1, body, 0)
            start_gather(0)
            start_gather(1)
            compute_scatter(0)
            compute_scatter(1)
            wait_scatter(0)
            wait_scatter(1)
            plsc.subcore_barrier()
            dump(out_h)

        @pl.when(c == 0)
        def _():
            one_pass(sc_h, dc_h, radc_h, True, aggc_h)

        @pl.when(c == 1)
        def _():
            one_pass(sn_h, dn_h, radn_h, False, aggn_h)

    return edge_k


def _radial_bond_tc(d2c, d2n, xb, wcc, bcc, wcn, bcn, wb, bb,
                    ec, en, dc, h_dim, be=2048):
    ep = d2c.shape[0]
    grid = (ep // be,)
    step_c = np.float32(6.0 / (dc - 1))
    step_n = np.float32(10.0 / (dc - 1))
    inv_sig_c = np.float32(dc / 6.0)
    inv_sig_n = np.float32(dc / 10.0)

    def body(d2c_ref, d2n_ref, xb_ref, wcc_ref, bcc_ref, wcn_ref, bcn_ref,
             wb_ref, bb_ref, rc_ref, rn_ref, bd_ref):
        e = pl.program_id(0)
        rows = e * be + lax.broadcasted_iota(jnp.int32, (be, 1), 0)

        def radial(d2_ref, w_ref, b_ref, mu_step, inv_sig, limit):
            mu = (lax.broadcasted_iota(jnp.int32, (1, dc), 1)
                  .astype(jnp.float32) * mu_step)
            d = jnp.sqrt(d2_ref[...] + 1e-12)
            t = (d - mu) * inv_sig
            rbf = jnp.exp(-(t * t))
            r = jnp.dot(rbf, w_ref[0], preferred_element_type=jnp.float32)
            r = _silu(r + b_ref[0])
            return jnp.where(rows < limit, r, 0.0)

        rc_ref[...] = radial(d2c_ref, wcc_ref, bcc_ref, step_c, inv_sig_c, ec)
        rn_ref[...] = radial(d2n_ref, wcn_ref, bcn_ref, step_n, inv_sig_n, en)
        bd_ref[...] = (jnp.dot(xb_ref[...], wb_ref[0],
                               preferred_element_type=jnp.float32)
                       + bb_ref[0])

    return pl.pallas_call(
        body,
        grid=grid,
        in_specs=[
            pl.BlockSpec((be, 1), lambda e: (e, 0)),
            pl.BlockSpec((be, 1), lambda e: (e, 0)),
            pl.BlockSpec((be, h_dim), lambda e: (e, 0)),
            pl.BlockSpec((1, dc, h_dim), lambda e: (0, 0, 0)),
            pl.BlockSpec((1, 1, h_dim), lambda e: (0, 0, 0)),
            pl.BlockSpec((1, dc, h_dim), lambda e: (0, 0, 0)),
            pl.BlockSpec((1, 1, h_dim), lambda e: (0, 0, 0)),
            pl.BlockSpec((1, h_dim, h_dim), lambda e: (0, 0, 0)),
            pl.BlockSpec((1, 1, h_dim), lambda e: (0, 0, 0)),
        ],
        out_specs=[
            pl.BlockSpec((be, h_dim), lambda e: (e, 0)),
            pl.BlockSpec((be, h_dim), lambda e: (e, 0)),
            pl.BlockSpec((be, h_dim), lambda e: (e, 0)),
        ],
        out_shape=[
            jax.ShapeDtypeStruct((ep, h_dim), jnp.float32),
            jax.ShapeDtypeStruct((ep, h_dim), jnp.float32),
            jax.ShapeDtypeStruct((ep, h_dim), jnp.float32),
        ],
    )(d2c, d2n, xb, wcc[None], bcc[None, None], wcn[None], bcn[None, None],
      wb[None], bb[None, None])


def _node_tc(h, aggc, aggn, wc, bc, wn, bn, x0, bn_rows=2000):
    n_nodes, h_dim = h.shape
    grid = (n_nodes // bn_rows,)

    def body(h_ref, ac_ref, an_ref, wc_ref, bc_ref, wn_ref, bn_ref, x0_ref,
             o_ref):
        hh = h_ref[...]
        zc = jnp.dot(hh + ac_ref[...], wc_ref[...],
                     preferred_element_type=jnp.float32) + bc_ref[...]
        zn = jnp.dot(hh + an_ref[...], wn_ref[...],
                     preferred_element_type=jnp.float32) + bn_ref[...]
        o_ref[...] = _lrelu(zc) + _lrelu(zn) + x0_ref[...]

    return pl.pallas_call(
        body,
        grid=grid,
        in_specs=[
            pl.BlockSpec((bn_rows, h_dim), lambda e: (e, 0)),
            pl.BlockSpec((bn_rows, h_dim), lambda e: (e, 0)),
            pl.BlockSpec((bn_rows, h_dim), lambda e: (e, 0)),
            pl.BlockSpec((h_dim, h_dim), lambda e: (0, 0)),
            pl.BlockSpec((1, h_dim), lambda e: (0, 0)),
            pl.BlockSpec((h_dim, h_dim), lambda e: (0, 0)),
            pl.BlockSpec((1, h_dim), lambda e: (0, 0)),
            pl.BlockSpec((bn_rows, h_dim), lambda e: (e, 0)),
        ],
        out_specs=pl.BlockSpec((bn_rows, h_dim), lambda e: (e, 0)),
        out_shape=jax.ShapeDtypeStruct((n_nodes, h_dim), jnp.float32),
    )(h, aggc, aggn, wc, bc[None], wn, bn[None], x0)


def kernel(init_x, x, x_bond, edge_index_intra, edge_index_inter, pos,
           W_coord_cov, b_coord_cov, W_coord_ncov, b_coord_ncov,
           W_bond, b_bond, W_node_cov, b_node_cov, W_node_ncov, b_node_ncov):
    n_nodes, h_dim = x.shape
    ec = edge_index_intra.shape[1]
    en = edge_index_inter.shape[1]
    n_layers = W_bond.shape[0]
    dc = W_coord_cov.shape[1]

    quant = NW * BC
    ep = ((max(ec, en) + quant - 1) // quant) * quant
    nq = NS * 8
    n_pad = ((n_nodes + nq - 1) // nq) * nq

    src_c = jnp.pad(edge_index_intra[0].astype(jnp.int32), (0, ep - ec))
    dst_c = jnp.pad(edge_index_intra[1].astype(jnp.int32), (0, ep - ec))
    src_n = jnp.pad(edge_index_inter[0].astype(jnp.int32), (0, ep - en))
    dst_n = jnp.pad(edge_index_inter[1].astype(jnp.int32), (0, ep - en))
    xb = jnp.pad(x_bond, ((0, ep - ec), (0, 0)))
    px = pos[:, 0] + 0.0
    py = pos[:, 1] + 0.0
    pz = pos[:, 2] + 0.0

    d2c, d2n = _build_dist_kernel(n_nodes, ep, ep)(
        px, py, pz, src_c, dst_c, src_n, dst_n)

    d2c2 = d2c.reshape(ep, 1)
    d2n2 = d2n.reshape(ep, 1)
    edge_k = _build_edge_kernel(n_pad, h_dim, ep, ep)

    h = x
    for i in range(n_layers):
        rc, rn, bd = _radial_bond_tc(
            d2c2, d2n2, xb,
            W_coord_cov[i], b_coord_cov[i], W_coord_ncov[i], b_coord_ncov[i],
            W_bond[i], b_bond[i], ec, en, dc, h_dim)
        aggc, aggn = edge_k(h, rc, rn, bd, src_c, dst_c, src_n, dst_n)
        h = _node_tc(h, aggc, aggn, W_node_cov[i], b_node_cov[i],
                     W_node_ncov[i], b_node_ncov[i], init_x)
    return h


# speedup vs baseline: 3.1277x; 1.0525x over previous
import functools

import numpy as np
import jax
import jax.numpy as jnp
from jax import lax
from jax.experimental import pallas as pl
from jax.experimental.pallas import tpu as pltpu
from jax.experimental.pallas import tpu_sc as plsc

NC = 2
NS = 16
NW = NC * NS
BC = 64


def _silu(v):
    return v / (1.0 + jnp.exp(-v))


def _lrelu(v):
    return jnp.where(v >= 0, v, 0.1 * v)


def _chunk(n, cap=128):
    for cand in range(min(n, cap), 0, -1):
        if n % cand == 0 and cand % 8 == 0:
            return cand
    return 8


def _build_dist_kernel(n_nodes, epc, epn):
    eac, ean = epc // NW, epn // NW
    ebuf = max(eac, ean)
    mesh = plsc.VectorSubcoreMesh(core_axis_name="c", subcore_axis_name="s")

    @functools.partial(
        pl.kernel,
        out_type=[jax.ShapeDtypeStruct((epc,), jnp.float32),
                  jax.ShapeDtypeStruct((epn,), jnp.float32)],
        mesh=mesh,
        scratch_types=[
            pltpu.VMEM((n_nodes,), jnp.float32),
            pltpu.VMEM((n_nodes,), jnp.float32),
            pltpu.VMEM((n_nodes,), jnp.float32),
            pltpu.VMEM((ebuf,), jnp.int32),
            pltpu.VMEM((ebuf,), jnp.int32),
            pltpu.VMEM((ebuf,), jnp.float32),
        ],
        compiler_params=pltpu.CompilerParams(needs_layout_passes=False),
    )
    def dist_k(px_h, py_h, pz_h, sc_h, dc_h, sn_h, dn_h, d2c_h, d2n_h,
               px_v, py_v, pz_v, s_v, t_v, d2_v):
        c = lax.axis_index("c")
        s = lax.axis_index("s")
        wid = s * NC + c
        pltpu.sync_copy(px_h, px_v)
        pltpu.sync_copy(py_h, py_v)
        pltpu.sync_copy(pz_h, pz_v)

        def one(src_h, dst_h, out_h, ea):
            base = wid * ea
            pltpu.sync_copy(src_h.at[pl.ds(base, ea)], s_v.at[pl.ds(0, ea)])
            pltpu.sync_copy(dst_h.at[pl.ds(base, ea)], t_v.at[pl.ds(0, ea)])

            def body(j, carry):
                sl = pl.ds(j * 16, 16)
                si = s_v[sl]
                di = t_v[sl]
                dx = plsc.load_gather(px_v, [di]) - plsc.load_gather(px_v, [si])
                dy = plsc.load_gather(py_v, [di]) - plsc.load_gather(py_v, [si])
                dz = plsc.load_gather(pz_v, [di]) - plsc.load_gather(pz_v, [si])
                d2_v[sl] = dx * dx + dy * dy + dz * dz
                return carry

            lax.fori_loop(0, ea // 16, body, 0)
            pltpu.sync_copy(d2_v.at[pl.ds(0, ea)], out_h.at[pl.ds(base, ea)])

        one(sc_h, dc_h, d2c_h, eac)
        one(sn_h, dn_h, d2n_h, ean)

    return dist_k


def _build_edge_kernel(n_pad, h_dim, epc, epn):
    epw = epc // NS
    nb = epw // BC
    assert epn == epc and nb % 2 == 0
    npt = n_pad // NS
    ndz = BC
    chunks = []
    o = 0
    while o < npt:
        chunks.append((o, min(ndz, npt - o)))
        o += ndz
    mesh = plsc.VectorSubcoreMesh(core_axis_name="c", subcore_axis_name="s")

    @functools.partial(
        pl.kernel,
        out_type=[jax.ShapeDtypeStruct((n_pad, h_dim), jnp.float32),
                  jax.ShapeDtypeStruct((n_pad, h_dim), jnp.float32)],
        mesh=mesh,
        scratch_types=[
            pltpu.VMEM_SHARED((n_pad, h_dim), jnp.float32),
            pltpu.VMEM((BC,), jnp.int32),
            pltpu.VMEM((BC,), jnp.int32),
            pltpu.VMEM((BC,), jnp.int32),
            pltpu.VMEM((BC,), jnp.int32),
            pltpu.VMEM((BC, h_dim), jnp.float32),
            pltpu.VMEM((BC, h_dim), jnp.float32),
            pltpu.VMEM((BC, h_dim // 2), jnp.float32),
            pltpu.VMEM((BC, h_dim // 2), jnp.float32),
            pltpu.VMEM((BC, h_dim // 2), jnp.float32),
            pltpu.VMEM((BC, h_dim // 2), jnp.float32),
            pltpu.SemaphoreType.DMA,
            pltpu.SemaphoreType.DMA,
            pltpu.SemaphoreType.DMA,
            pltpu.SemaphoreType.DMA,
            pltpu.SemaphoreType.DMA,
            pltpu.SemaphoreType.DMA,
            pltpu.SemaphoreType.DMA,
            pltpu.SemaphoreType.DMA,
            pltpu.SemaphoreType.DMA,
            pltpu.SemaphoreType.DMA,
        ],
        compiler_params=pltpu.CompilerParams(needs_layout_passes=False),
    )
    def edge_k(h_hbm, radc_h, radn_h, bond_h, sc_h, dc_h, sn_h, dn_h,
               aggc_h, aggn_h,
               agg_sh, si0, si1, di0, di1, rows0, rows1, rad0, rad1,
               bond0, bond1,
               smi0, smi1, smb0, smb1, smr0, smr1, smg0, smg1, sms0, sms1):
        bnc = rows0
        c = lax.axis_index("c")
        s = lax.axis_index("s")
        base = s * epw
        row0 = s * npt
        si = (si0, si1)
        di = (di0, di1)
        rows = (rows0, rows1)
        rad = (rad0, rad1)
        bond = (bond0, bond1)
        smi = (smi0, smi1)
        smb = (smb0, smb1)
        smr = (smr0, smr1)
        smg = (smg0, smg1)
        sms = (sms0, sms1)

        def zero_bounce():
            def zb(r, carry):
                for g in range(h_dim // 16):
                    bnc[r, pl.ds(g * 16, 16)] = jnp.zeros((16,), jnp.float32)
                return carry
            lax.fori_loop(0, ndz, zb, 0)

        def zero_agg():
            for (ro, rn) in chunks:
                pltpu.sync_copy(bnc.at[pl.ds(0, rn), :],
                                agg_sh.at[pl.ds(row0 + ro, rn), :])

        def dump(out_h):
            for (ro, rn) in chunks:
                pltpu.sync_copy(agg_sh.at[pl.ds(row0 + ro, rn), :],
                                bnc.at[pl.ds(0, rn), :])
                pltpu.sync_copy(bnc.at[pl.ds(0, rn), :],
                                out_h.at[pl.ds(row0 + ro, rn), :])

        def one_pass(src_h, dst_h, rad_h, use_bond, out_h):
            def issue_loads(p, b):
                off = base + b * BC
                pltpu.async_copy(src_h.at[pl.ds(off, BC)], si[p], smi[p])
                pltpu.async_copy(dst_h.at[pl.ds(off, BC)], di[p], smi[p])
                if use_bond:
                    pltpu.async_copy(bond_h.at[pl.ds(off, BC), :],
                                     bond[p], smb[p])
                pltpu.async_copy(rad_h.at[pl.ds(off, BC), :],
                                 rad[p], smr[p])

            def wait_idx(p):
                pltpu.make_async_copy(src_h.at[pl.ds(0, BC)], si[p],
                                      smi[p]).wait()
                pltpu.make_async_copy(dst_h.at[pl.ds(0, BC)], di[p],
                                      smi[p]).wait()

            def start_gather(p):
                wait_idx(p)
                pltpu.async_copy(h_hbm.at[si[p]], rows[p], smg[p])

            def compute_scatter(p):
                pltpu.make_async_copy(h_hbm.at[si[p]], rows[p],
                                      smg[p]).wait()
                pltpu.make_async_copy(rad_h.at[pl.ds(0, BC), :], rad[p],
                                      smr[p]).wait()
                if use_bond:
                    pltpu.make_async_copy(bond_h.at[pl.ds(0, BC), :],
                                          bond[p], smb[p]).wait()

                def rb(r, cc):
                    for g in range(h_dim // 32):
                        sa = pl.ds(g * 16, 16)
                        sb = pl.ds(h_dim // 2 + g * 16, 16)
                        ra, rb2 = plsc.unpack(
                            plsc.bitcast(rad[p][r, pl.ds(g * 16, 16)],
                                         jnp.bfloat16),
                            format=plsc.PackFormat.INTERLEAVED)
                        va = rows[p][r, sa]
                        vb = rows[p][r, sb]
                        if use_bond:
                            ba, bb2 = plsc.unpack(
                                plsc.bitcast(bond[p][r, pl.ds(g * 16, 16)],
                                             jnp.bfloat16),
                                format=plsc.PackFormat.INTERLEAVED)
                            va = va + ba
                            vb = vb + bb2
                        rows[p][r, sa] = va * ra
                        rows[p][r, sb] = vb * rb2
                    return cc

                lax.fori_loop(0, BC, rb, 0)
                pltpu.async_copy(rows[p], agg_sh.at[di[p]], sms[p], add=True)

            def wait_scatter(p):
                pltpu.make_async_copy(rows[p], agg_sh.at[di[p]],
                                      sms[p]).wait()

            zero_bounce()
            zero_agg()
            issue_loads(0, 0)
            issue_loads(1, 1)
            plsc.subcore_barrier()

            def body(k, carry):
                a = 2 * k
                start_gather(0)
                start_gather(1)
                compute_scatter(0)
                compute_scatter(1)
                wait_scatter(0)
                issue_loads(0, a + 2)
                wait_scatter(1)
                issue_loads(1, a + 3)
                return carry

            lax.fori_loop(0, nb // 2 - 1, body, 0)
            start_gather(0)
            start_gather(1)
            compute_scatter(0)
            compute_scatter(1)
            wait_scatter(0)
            wait_scatter(1)
            plsc.subcore_barrier()
            dump(out_h)

        @pl.when(c == 0)
        def _():
            one_pass(sc_h, dc_h, radc_h, True, aggc_h)

        @pl.when(c == 1)
        def _():
            one_pass(sn_h, dn_h, radn_h, False, aggn_h)

    return edge_k


def _radial_bond_tc(d2c, d2n, xb, wcc, bcc, wcn, bcn, wb, bb,
                    ec, en, dc, h_dim, be=2048):
    ep = d2c.shape[0]
    grid = (ep // be,)
    step_c = np.float32(6.0 / (dc - 1))
    step_n = np.float32(10.0 / (dc - 1))
    inv_sig_c = np.float32(dc / 6.0)
    inv_sig_n = np.float32(dc / 10.0)

    def body(d2c_ref, d2n_ref, xb_ref, wcc_ref, bcc_ref, wcn_ref, bcn_ref,
             wb_ref, bb_ref, rc_ref, rn_ref, bd_ref):
        e = pl.program_id(0)
        rows = e * be + lax.broadcasted_iota(jnp.int32, (be, 1), 0)

        def radial(d2_ref, w_ref, b_ref, mu_step, inv_sig, limit):
            mu = (lax.broadcasted_iota(jnp.int32, (1, dc), 1)
                  .astype(jnp.float32) * mu_step)
            d = jnp.sqrt(d2_ref[...] + 1e-12)
            t = (d - mu) * inv_sig
            rbf = jnp.exp(-(t * t))
            r = jnp.dot(rbf, w_ref[0], preferred_element_type=jnp.float32)
            r = _silu(r + b_ref[0])
            return jnp.where(rows < limit, r, 0.0)

        def pack_words(v):
            u = jax.lax.bitcast_convert_type(v, jnp.uint32)
            rne = u + jnp.uint32(0x7FFF) + ((u >> 16) & jnp.uint32(1))
            lo = rne[:, :h_dim // 2] >> 16
            hi = rne[:, h_dim // 2:] & jnp.uint32(0xFFFF0000)
            return jax.lax.bitcast_convert_type(lo | hi, jnp.float32)

        rc_ref[...] = pack_words(
            radial(d2c_ref, wcc_ref, bcc_ref, step_c, inv_sig_c, ec))
        rn_ref[...] = pack_words(
            radial(d2n_ref, wcn_ref, bcn_ref, step_n, inv_sig_n, en))
        bd_ref[...] = pack_words(
            jnp.dot(xb_ref[...], wb_ref[0],
                    preferred_element_type=jnp.float32) + bb_ref[0])

    return pl.pallas_call(
        body,
        grid=grid,
        in_specs=[
            pl.BlockSpec((be, 1), lambda e: (e, 0)),
            pl.BlockSpec((be, 1), lambda e: (e, 0)),
            pl.BlockSpec((be, h_dim), lambda e: (e, 0)),
            pl.BlockSpec((1, dc, h_dim), lambda e: (0, 0, 0)),
            pl.BlockSpec((1, 1, h_dim), lambda e: (0, 0, 0)),
            pl.BlockSpec((1, dc, h_dim), lambda e: (0, 0, 0)),
            pl.BlockSpec((1, 1, h_dim), lambda e: (0, 0, 0)),
            pl.BlockSpec((1, h_dim, h_dim), lambda e: (0, 0, 0)),
            pl.BlockSpec((1, 1, h_dim), lambda e: (0, 0, 0)),
        ],
        out_specs=[
            pl.BlockSpec((be, h_dim // 2), lambda e: (e, 0)),
            pl.BlockSpec((be, h_dim // 2), lambda e: (e, 0)),
            pl.BlockSpec((be, h_dim // 2), lambda e: (e, 0)),
        ],
        out_shape=[
            jax.ShapeDtypeStruct((ep, h_dim // 2), jnp.float32),
            jax.ShapeDtypeStruct((ep, h_dim // 2), jnp.float32),
            jax.ShapeDtypeStruct((ep, h_dim // 2), jnp.float32),
        ],
    )(d2c, d2n, xb, wcc[None], bcc[None, None], wcn[None], bcn[None, None],
      wb[None], bb[None, None])


def _node_tc(h, aggc, aggn, wc, bc, wn, bn, x0, bn_rows=2000):
    n_nodes, h_dim = h.shape
    grid = (n_nodes // bn_rows,)

    def body(h_ref, ac_ref, an_ref, wc_ref, bc_ref, wn_ref, bn_ref, x0_ref,
             o_ref):
        hh = h_ref[...]
        zc = jnp.dot(hh + ac_ref[...], wc_ref[...],
                     preferred_element_type=jnp.float32) + bc_ref[...]
        zn = jnp.dot(hh + an_ref[...], wn_ref[...],
                     preferred_element_type=jnp.float32) + bn_ref[...]
        o_ref[...] = _lrelu(zc) + _lrelu(zn) + x0_ref[...]

    return pl.pallas_call(
        body,
        grid=grid,
        in_specs=[
            pl.BlockSpec((bn_rows, h_dim), lambda e: (e, 0)),
            pl.BlockSpec((bn_rows, h_dim), lambda e: (e, 0)),
            pl.BlockSpec((bn_rows, h_dim), lambda e: (e, 0)),
            pl.BlockSpec((h_dim, h_dim), lambda e: (0, 0)),
            pl.BlockSpec((1, h_dim), lambda e: (0, 0)),
            pl.BlockSpec((h_dim, h_dim), lambda e: (0, 0)),
            pl.BlockSpec((1, h_dim), lambda e: (0, 0)),
            pl.BlockSpec((bn_rows, h_dim), lambda e: (e, 0)),
        ],
        out_specs=pl.BlockSpec((bn_rows, h_dim), lambda e: (e, 0)),
        out_shape=jax.ShapeDtypeStruct((n_nodes, h_dim), jnp.float32),
    )(h, aggc, aggn, wc, bc[None], wn, bn[None], x0)


def kernel(init_x, x, x_bond, edge_index_intra, edge_index_inter, pos,
           W_coord_cov, b_coord_cov, W_coord_ncov, b_coord_ncov,
           W_bond, b_bond, W_node_cov, b_node_cov, W_node_ncov, b_node_ncov):
    n_nodes, h_dim = x.shape
    ec = edge_index_intra.shape[1]
    en = edge_index_inter.shape[1]
    n_layers = W_bond.shape[0]
    dc = W_coord_cov.shape[1]

    quant = NW * BC
    ep = ((max(ec, en) + quant - 1) // quant) * quant
    nq = NS * 8
    n_pad = ((n_nodes + nq - 1) // nq) * nq

    src_c = jnp.pad(edge_index_intra[0].astype(jnp.int32), (0, ep - ec))
    dst_c = jnp.pad(edge_index_intra[1].astype(jnp.int32), (0, ep - ec))
    src_n = jnp.pad(edge_index_inter[0].astype(jnp.int32), (0, ep - en))
    dst_n = jnp.pad(edge_index_inter[1].astype(jnp.int32), (0, ep - en))
    xb = jnp.pad(x_bond, ((0, ep - ec), (0, 0)))
    px = pos[:, 0] + 0.0
    py = pos[:, 1] + 0.0
    pz = pos[:, 2] + 0.0

    d2c, d2n = _build_dist_kernel(n_nodes, ep, ep)(
        px, py, pz, src_c, dst_c, src_n, dst_n)

    d2c2 = d2c.reshape(ep, 1)
    d2n2 = d2n.reshape(ep, 1)
    edge_k = _build_edge_kernel(n_pad, h_dim, ep, ep)

    h = x
    for i in range(n_layers):
        rc, rn, bd = _radial_bond_tc(
            d2c2, d2n2, xb,
            W_coord_cov[i], b_coord_cov[i], W_coord_ncov[i], b_coord_ncov[i],
            W_bond[i], b_bond[i], ec, en, dc, h_dim)
        aggc, aggn = edge_k(h, rc, rn, bd, src_c, dst_c, src_n, dst_n)
        h = _node_tc(h, aggc, aggn, W_node_cov[i], b_node_cov[i],
                     W_node_ncov[i], b_node_ncov[i], init_x)
    return h


# speedup vs baseline: 3.2257x; 1.0313x over previous
import functools

import numpy as np
import jax
import jax.numpy as jnp
from jax import lax
from jax.experimental import pallas as pl
from jax.experimental.pallas import tpu as pltpu
from jax.experimental.pallas import tpu_sc as plsc

NC = 2
NS = 16
NW = NC * NS
BC = 64


def _silu(v):
    return v / (1.0 + jnp.exp(-v))


def _lrelu(v):
    return jnp.where(v >= 0, v, 0.1 * v)


def _chunk(n, cap=128):
    for cand in range(min(n, cap), 0, -1):
        if n % cand == 0 and cand % 8 == 0:
            return cand
    return 8


def _build_dist_kernel(n_nodes, epc, epn):
    eac, ean = epc // NW, epn // NW
    ebuf = max(eac, ean)
    mesh = plsc.VectorSubcoreMesh(core_axis_name="c", subcore_axis_name="s")

    @functools.partial(
        pl.kernel,
        out_type=[jax.ShapeDtypeStruct((epc,), jnp.float32),
                  jax.ShapeDtypeStruct((epn,), jnp.float32)],
        mesh=mesh,
        scratch_types=[
            pltpu.VMEM((n_nodes,), jnp.float32),
            pltpu.VMEM((n_nodes,), jnp.float32),
            pltpu.VMEM((n_nodes,), jnp.float32),
            pltpu.VMEM((ebuf,), jnp.int32),
            pltpu.VMEM((ebuf,), jnp.int32),
            pltpu.VMEM((ebuf,), jnp.float32),
        ],
        compiler_params=pltpu.CompilerParams(needs_layout_passes=False),
    )
    def dist_k(px_h, py_h, pz_h, sc_h, dc_h, sn_h, dn_h, d2c_h, d2n_h,
               px_v, py_v, pz_v, s_v, t_v, d2_v):
        c = lax.axis_index("c")
        s = lax.axis_index("s")
        wid = s * NC + c
        pltpu.sync_copy(px_h, px_v)
        pltpu.sync_copy(py_h, py_v)
        pltpu.sync_copy(pz_h, pz_v)

        def one(src_h, dst_h, out_h, ea):
            base = wid * ea
            pltpu.sync_copy(src_h.at[pl.ds(base, ea)], s_v.at[pl.ds(0, ea)])
            pltpu.sync_copy(dst_h.at[pl.ds(base, ea)], t_v.at[pl.ds(0, ea)])

            def body(j, carry):
                sl = pl.ds(j * 16, 16)
                si = s_v[sl]
                di = t_v[sl]
                dx = plsc.load_gather(px_v, [di]) - plsc.load_gather(px_v, [si])
                dy = plsc.load_gather(py_v, [di]) - plsc.load_gather(py_v, [si])
                dz = plsc.load_gather(pz_v, [di]) - plsc.load_gather(pz_v, [si])
                d2_v[sl] = dx * dx + dy * dy + dz * dz
                return carry

            lax.fori_loop(0, ea // 16, body, 0)
            pltpu.sync_copy(d2_v.at[pl.ds(0, ea)], out_h.at[pl.ds(base, ea)])

        one(sc_h, dc_h, d2c_h, eac)
        one(sn_h, dn_h, d2n_h, ean)

    return dist_k


def _build_edge_kernel(n_pad, h_dim, epc, epn):
    epw = epc // NS
    nb = epw // BC
    assert epn == epc and nb % 2 == 0
    npt = n_pad // NS
    ndz = BC
    chunks = []
    o = 0
    while o < npt:
        chunks.append((o, min(ndz, npt - o)))
        o += ndz
    mesh = plsc.VectorSubcoreMesh(core_axis_name="c", subcore_axis_name="s")

    @functools.partial(
        pl.kernel,
        out_type=[jax.ShapeDtypeStruct((n_pad, h_dim), jnp.float32),
                  jax.ShapeDtypeStruct((n_pad, h_dim), jnp.float32)],
        mesh=mesh,
        scratch_types=[
            pltpu.VMEM_SHARED((n_pad, h_dim), jnp.float32),
            pltpu.VMEM((BC,), jnp.int32),
            pltpu.VMEM((BC,), jnp.int32),
            pltpu.VMEM((BC,), jnp.int32),
            pltpu.VMEM((BC,), jnp.int32),
            pltpu.VMEM((BC,), jnp.int32),
            pltpu.VMEM((BC,), jnp.int32),
            pltpu.VMEM((BC,), jnp.int32),
            pltpu.VMEM((BC,), jnp.int32),
            pltpu.VMEM((BC, h_dim), jnp.float32),
            pltpu.VMEM((BC, h_dim), jnp.float32),
            pltpu.VMEM((BC, h_dim // 2), jnp.float32),
            pltpu.VMEM((BC, h_dim // 2), jnp.float32),
            pltpu.VMEM((BC, h_dim // 2), jnp.float32),
            pltpu.VMEM((BC, h_dim // 2), jnp.float32),
            pltpu.SemaphoreType.DMA,
            pltpu.SemaphoreType.DMA,
            pltpu.SemaphoreType.DMA,
            pltpu.SemaphoreType.DMA,
            pltpu.SemaphoreType.DMA,
            pltpu.SemaphoreType.DMA,
            pltpu.SemaphoreType.DMA,
            pltpu.SemaphoreType.DMA,
            pltpu.SemaphoreType.DMA,
            pltpu.SemaphoreType.DMA,
            pltpu.SemaphoreType.DMA,
            pltpu.SemaphoreType.DMA,
        ],
        compiler_params=pltpu.CompilerParams(needs_layout_passes=False),
    )
    def edge_k(h_hbm, radc_h, radn_h, bond_h, sc_h, dc_h, sn_h, dn_h,
               aggc_h, aggn_h,
               agg_sh, si0, si1, si2, si3, di0, di1, di2, di3,
               rows0, rows1, rad0, rad1, bond0, bond1,
               smi0, smi1, smi2, smi3, smb0, smb1, smr0, smr1,
               smg0, smg1, sms0, sms1):
        bnc = rows0
        c = lax.axis_index("c")
        s = lax.axis_index("s")
        base = s * epw
        row0 = s * npt
        si = (si0, si1, si2, si3)
        di = (di0, di1, di2, di3)
        rows = (rows0, rows1)
        rad = (rad0, rad1)
        bond = (bond0, bond1)
        smi = (smi0, smi1, smi2, smi3)
        smb = (smb0, smb1)
        smr = (smr0, smr1)
        smg = (smg0, smg1)
        sms = (sms0, sms1)

        def zero_bounce():
            def zb(r, carry):
                for g in range(h_dim // 16):
                    bnc[r, pl.ds(g * 16, 16)] = jnp.zeros((16,), jnp.float32)
                return carry
            lax.fori_loop(0, ndz, zb, 0)

        def zero_agg():
            for (ro, rn) in chunks:
                pltpu.sync_copy(bnc.at[pl.ds(0, rn), :],
                                agg_sh.at[pl.ds(row0 + ro, rn), :])

        def dump(out_h):
            for (ro, rn) in chunks:
                pltpu.sync_copy(agg_sh.at[pl.ds(row0 + ro, rn), :],
                                bnc.at[pl.ds(0, rn), :])
                pltpu.sync_copy(bnc.at[pl.ds(0, rn), :],
                                out_h.at[pl.ds(row0 + ro, rn), :])

        def one_pass(src_h, dst_h, rad_h, use_bond, out_h):
            def issue_idx(q, b):
                off = base + b * BC
                pltpu.async_copy(src_h.at[pl.ds(off, BC)], si[q], smi[q])
                pltpu.async_copy(dst_h.at[pl.ds(off, BC)], di[q], smi[q])

            def wait_idx(q):
                pltpu.make_async_copy(src_h.at[pl.ds(0, BC)], si[q],
                                      smi[q]).wait()
                pltpu.make_async_copy(dst_h.at[pl.ds(0, BC)], di[q],
                                      smi[q]).wait()

            def issue_gather(p, q):
                wait_idx(q)
                pltpu.async_copy(h_hbm.at[si[q]], rows[p], smg[p])

            def issue_rb(p, b):
                off = base + b * BC
                if use_bond:
                    pltpu.async_copy(bond_h.at[pl.ds(off, BC), :],
                                     bond[p], smb[p])
                pltpu.async_copy(rad_h.at[pl.ds(off, BC), :],
                                 rad[p], smr[p])

            def compute_scatter(p, q):
                pltpu.make_async_copy(h_hbm.at[si[q]], rows[p],
                                      smg[p]).wait()
                pltpu.make_async_copy(rad_h.at[pl.ds(0, BC), :], rad[p],
                                      smr[p]).wait()
                if use_bond:
                    pltpu.make_async_copy(bond_h.at[pl.ds(0, BC), :],
                                          bond[p], smb[p]).wait()

                def one_row(r):
                    for g in range(h_dim // 32):
                        sa = pl.ds(g * 16, 16)
                        sb = pl.ds(h_dim // 2 + g * 16, 16)
                        ra, rb2 = plsc.unpack(
                            plsc.bitcast(rad[p][r, pl.ds(g * 16, 16)],
                                         jnp.bfloat16),
                            format=plsc.PackFormat.INTERLEAVED)
                        va = rows[p][r, sa]
                        vb = rows[p][r, sb]
                        if use_bond:
                            ba, bb2 = plsc.unpack(
                                plsc.bitcast(bond[p][r, pl.ds(g * 16, 16)],
                                             jnp.bfloat16),
                                format=plsc.PackFormat.INTERLEAVED)
                            va = va + ba
                            vb = vb + bb2
                        rows[p][r, sa] = va * ra
                        rows[p][r, sb] = vb * rb2

                def rb(r2, cc):
                    one_row(2 * r2)
                    one_row(2 * r2 + 1)
                    return cc

                lax.fori_loop(0, BC // 2, rb, 0)
                pltpu.async_copy(rows[p], agg_sh.at[di[q]], sms[p], add=True)

            def wait_scatter(p, q):
                pltpu.make_async_copy(rows[p], agg_sh.at[di[q]],
                                      sms[p]).wait()

            zero_bounce()
            zero_agg()
            for q in range(4):
                issue_idx(q, q)
            issue_gather(0, 0)
            issue_rb(0, 0)
            issue_gather(1, 1)
            issue_rb(1, 1)
            plsc.subcore_barrier()

            def body(k, carry):
                a = 4 * k
                compute_scatter(0, 0)
                compute_scatter(1, 1)
                wait_scatter(0, 0)
                issue_idx(0, a + 4)
                issue_gather(0, 2)
                issue_rb(0, a + 2)
                wait_scatter(1, 1)
                issue_idx(1, a + 5)
                issue_gather(1, 3)
                issue_rb(1, a + 3)
                compute_scatter(0, 2)
                compute_scatter(1, 3)
                wait_scatter(0, 2)
                issue_idx(2, a + 6)
                issue_gather(0, 0)
                issue_rb(0, a + 4)
                wait_scatter(1, 3)
                issue_idx(3, a + 7)
                issue_gather(1, 1)
                issue_rb(1, a + 5)
                return carry

            lax.fori_loop(0, nb // 4 - 1, body, 0)
            compute_scatter(0, 0)
            compute_scatter(1, 1)
            wait_scatter(0, 0)
            issue_gather(0, 2)
            issue_rb(0, nb - 2)
            wait_scatter(1, 1)
            issue_gather(1, 3)
            issue_rb(1, nb - 1)
            compute_scatter(0, 2)
            compute_scatter(1, 3)
            wait_scatter(0, 2)
            wait_scatter(1, 3)
            plsc.subcore_barrier()
            dump(out_h)

        @pl.when(c == 0)
        def _():
            one_pass(sc_h, dc_h, radc_h, True, aggc_h)

        @pl.when(c == 1)
        def _():
            one_pass(sn_h, dn_h, radn_h, False, aggn_h)

    return edge_k


def _radial_bond_tc(d2c, d2n, xb, wcc, bcc, wcn, bcn, wb, bb,
                    ec, en, dc, h_dim, be=2048):
    ep = d2c.shape[0]
    grid = (ep // be,)
    step_c = np.float32(6.0 / (dc - 1))
    step_n = np.float32(10.0 / (dc - 1))
    inv_sig_c = np.float32(dc / 6.0)
    inv_sig_n = np.float32(dc / 10.0)

    def body(d2c_ref, d2n_ref, xb_ref, wcc_ref, bcc_ref, wcn_ref, bcn_ref,
             wb_ref, bb_ref, rc_ref, rn_ref, bd_ref):
        e = pl.program_id(0)
        rows = e * be + lax.broadcasted_iota(jnp.int32, (be, 1), 0)

        def radial(d2_ref, w_ref, b_ref, mu_step, inv_sig, limit):
            mu = (lax.broadcasted_iota(jnp.int32, (1, dc), 1)
                  .astype(jnp.float32) * mu_step)
            d = jnp.sqrt(d2_ref[...] + 1e-12)
            t = (d - mu) * inv_sig
            rbf = jnp.exp(-(t * t))
            r = jnp.dot(rbf, w_ref[0], preferred_element_type=jnp.float32)
            r = _silu(r + b_ref[0])
            return jnp.where(rows < limit, r, 0.0)

        def pack_words(v):
            u = jax.lax.bitcast_convert_type(v, jnp.uint32)
            rne = u + jnp.uint32(0x7FFF) + ((u >> 16) & jnp.uint32(1))
            lo = rne[:, :h_dim // 2] >> 16
            hi = rne[:, h_dim // 2:] & jnp.uint32(0xFFFF0000)
            return jax.lax.bitcast_convert_type(lo | hi, jnp.float32)

        rc_ref[...] = pack_words(
            radial(d2c_ref, wcc_ref, bcc_ref, step_c, inv_sig_c, ec))
        rn_ref[...] = pack_words(
            radial(d2n_ref, wcn_ref, bcn_ref, step_n, inv_sig_n, en))
        bd_ref[...] = pack_words(
            jnp.dot(xb_ref[...], wb_ref[0],
                    preferred_element_type=jnp.float32) + bb_ref[0])

    return pl.pallas_call(
        body,
        grid=grid,
        in_specs=[
            pl.BlockSpec((be, 1), lambda e: (e, 0)),
            pl.BlockSpec((be, 1), lambda e: (e, 0)),
            pl.BlockSpec((be, h_dim), lambda e: (e, 0)),
            pl.BlockSpec((1, dc, h_dim), lambda e: (0, 0, 0)),
            pl.BlockSpec((1, 1, h_dim), lambda e: (0, 0, 0)),
            pl.BlockSpec((1, dc, h_dim), lambda e: (0, 0, 0)),
            pl.BlockSpec((1, 1, h_dim), lambda e: (0, 0, 0)),
            pl.BlockSpec((1, h_dim, h_dim), lambda e: (0, 0, 0)),
            pl.BlockSpec((1, 1, h_dim), lambda e: (0, 0, 0)),
        ],
        out_specs=[
            pl.BlockSpec((be, h_dim // 2), lambda e: (e, 0)),
            pl.BlockSpec((be, h_dim // 2), lambda e: (e, 0)),
            pl.BlockSpec((be, h_dim // 2), lambda e: (e, 0)),
        ],
        out_shape=[
            jax.ShapeDtypeStruct((ep, h_dim // 2), jnp.float32),
            jax.ShapeDtypeStruct((ep, h_dim // 2), jnp.float32),
            jax.ShapeDtypeStruct((ep, h_dim // 2), jnp.float32),
        ],
    )(d2c, d2n, xb, wcc[None], bcc[None, None], wcn[None], bcn[None, None],
      wb[None], bb[None, None])


def _node_tc(h, aggc, aggn, wc, bc, wn, bn, x0, bn_rows=2000):
    n_nodes, h_dim = h.shape
    grid = (n_nodes // bn_rows,)

    def body(h_ref, ac_ref, an_ref, wc_ref, bc_ref, wn_ref, bn_ref, x0_ref,
             o_ref):
        hh = h_ref[...]
        zc = jnp.dot(hh + ac_ref[...], wc_ref[...],
                     preferred_element_type=jnp.float32) + bc_ref[...]
        zn = jnp.dot(hh + an_ref[...], wn_ref[...],
                     preferred_element_type=jnp.float32) + bn_ref[...]
        o_ref[...] = _lrelu(zc) + _lrelu(zn) + x0_ref[...]

    return pl.pallas_call(
        body,
        grid=grid,
        in_specs=[
            pl.BlockSpec((bn_rows, h_dim), lambda e: (e, 0)),
            pl.BlockSpec((bn_rows, h_dim), lambda e: (e, 0)),
            pl.BlockSpec((bn_rows, h_dim), lambda e: (e, 0)),
            pl.BlockSpec((h_dim, h_dim), lambda e: (0, 0)),
            pl.BlockSpec((1, h_dim), lambda e: (0, 0)),
            pl.BlockSpec((h_dim, h_dim), lambda e: (0, 0)),
            pl.BlockSpec((1, h_dim), lambda e: (0, 0)),
            pl.BlockSpec((bn_rows, h_dim), lambda e: (e, 0)),
        ],
        out_specs=pl.BlockSpec((bn_rows, h_dim), lambda e: (e, 0)),
        out_shape=jax.ShapeDtypeStruct((n_nodes, h_dim), jnp.float32),
    )(h, aggc, aggn, wc, bc[None], wn, bn[None], x0)


def kernel(init_x, x, x_bond, edge_index_intra, edge_index_inter, pos,
           W_coord_cov, b_coord_cov, W_coord_ncov, b_coord_ncov,
           W_bond, b_bond, W_node_cov, b_node_cov, W_node_ncov, b_node_ncov):
    n_nodes, h_dim = x.shape
    ec = edge_index_intra.shape[1]
    en = edge_index_inter.shape[1]
    n_layers = W_bond.shape[0]
    dc = W_coord_cov.shape[1]

    quant = NW * BC
    ep = ((max(ec, en) + quant - 1) // quant) * quant
    nq = NS * 8
    n_pad = ((n_nodes + nq - 1) // nq) * nq

    src_c = jnp.pad(edge_index_intra[0].astype(jnp.int32), (0, ep - ec))
    dst_c = jnp.pad(edge_index_intra[1].astype(jnp.int32), (0, ep - ec))
    src_n = jnp.pad(edge_index_inter[0].astype(jnp.int32), (0, ep - en))
    dst_n = jnp.pad(edge_index_inter[1].astype(jnp.int32), (0, ep - en))
    xb = jnp.pad(x_bond, ((0, ep - ec), (0, 0)))
    px = pos[:, 0] + 0.0
    py = pos[:, 1] + 0.0
    pz = pos[:, 2] + 0.0

    d2c, d2n = _build_dist_kernel(n_nodes, ep, ep)(
        px, py, pz, src_c, dst_c, src_n, dst_n)

    d2c2 = d2c.reshape(ep, 1)
    d2n2 = d2n.reshape(ep, 1)
    edge_k = _build_edge_kernel(n_pad, h_dim, ep, ep)

    h = x
    for i in range(n_layers):
        rc, rn, bd = _radial_bond_tc(
            d2c2, d2n2, xb,
            W_coord_cov[i], b_coord_cov[i], W_coord_ncov[i], b_coord_ncov[i],
            W_bond[i], b_bond[i], ec, en, dc, h_dim)
        aggc, aggn = edge_k(h, rc, rn, bd, src_c, dst_c, src_n, dst_n)
        h = _node_tc(h, aggc, aggn, W_node_cov[i], b_node_cov[i],
                     W_node_ncov[i], b_node_ncov[i], init_x)
    return h
